# baseline (speedup 1.0000x reference)
import functools

import jax
import jax.numpy as jnp
from jax import lax
from jax.experimental import pallas as pl
from jax.experimental.pallas import tpu as pltpu
from jax.experimental.pallas import tpu_sc as plsc

N_NODES = 10000
N_EDGES = 320000
D = 128

NC = 2
NS = 16
NW = NC * NS
E_TILE = N_EDGES // NW
E_CH = 80
N_CH = E_TILE // E_CH
N_PAD = 10240
ROWS_TILE = N_PAD // NS
ST_LEN = 10112
E_FLAT = 2 * N_EDGES + 128


def _fill_f32(ref, rows, cols, value):
    per_row = cols // 16

    def body(t, carry):
        i = t // per_row
        j = t % per_row
        ref[i, pl.ds(j * 16, 16)] = jnp.full((16,), value, jnp.float32)
        return carry

    lax.fori_loop(0, rows * per_row, body, 0)


def _sc_aggregate(features, edges1d):
    mesh = plsc.VectorSubcoreMesh(core_axis_name="c", subcore_axis_name="s")

    @functools.partial(
        pl.kernel,
        mesh=mesh,
        out_type=[
            jax.ShapeDtypeStruct((NC, N_NODES, D), jnp.float32),
            jax.ShapeDtypeStruct((NC, N_PAD), jnp.float32),
        ],
        scratch_types=[
            pltpu.VMEM((ST_LEN,), jnp.int32),
            pltpu.VMEM((ST_LEN,), jnp.int32),
            pltpu.VMEM((2, E_CH, D), jnp.float32),
            pltpu.VMEM((E_CH,), jnp.float32),
            pltpu.VMEM_SHARED((N_PAD, D), jnp.float32),
            pltpu.VMEM_SHARED((N_PAD,), jnp.float32),
            pltpu.SemaphoreType.DMA,
            pltpu.SemaphoreType.DMA,
            pltpu.SemaphoreType.DMA,
            pltpu.SemaphoreType.DMA,
        ],
    )
    def agg(features_hbm, edges_hbm, psum_hbm, pcnt_hbm,
            src_v, dst_v, rows2, ones_v, acc, cacc,
            sem_ga, sem_gb, sem_sa, sem_sb):
        cid = lax.axis_index("c")
        sid = lax.axis_index("s")
        gid = cid * NS + sid
        base = sid * ROWS_TILE

        zbuf = rows2.at[0]
        _fill_f32(zbuf, E_CH, D, 0.0)
        for k in range(ROWS_TILE // E_CH):
            pltpu.sync_copy(zbuf, acc.at[pl.ds(base + k * E_CH, E_CH)])
        for k in range(ROWS_TILE // D):
            pltpu.sync_copy(zbuf.at[0], cacc.at[pl.ds(base + k * D, D)])

        def fill_ones(j, carry):
            ones_v[pl.ds(j * 16, 16)] = jnp.full((16,), 1.0, jnp.float32)
            return carry

        lax.fori_loop(0, E_CH // 16, fill_ones, 0)

        pltpu.sync_copy(edges_hbm.at[pl.ds(gid * E_TILE, ST_LEN)], src_v)
        pltpu.sync_copy(edges_hbm.at[pl.ds(N_EDGES + gid * E_TILE, ST_LEN)],
                        dst_v)

        plsc.subcore_barrier()

        gsem = (sem_ga, sem_gb)

        def gather(c, buf):
            pltpu.async_copy(
                features_hbm.at[src_v.at[pl.ds(c * E_CH, E_CH)]],
                rows2.at[buf], gsem[buf])

        def drain_g(buf):
            pltpu.make_async_copy(
                features_hbm.at[src_v.at[pl.ds(0, E_CH)]],
                rows2.at[buf], gsem[buf]).wait()

        def scatter(c, buf):
            idx = dst_v.at[pl.ds(c * E_CH, E_CH)]
            pltpu.sync_copy(rows2.at[buf], acc.at[idx], add=True)
            pltpu.async_copy(ones_v, cacc.at[idx], sem_sa, add=True)

        def drain_ones():
            pltpu.make_async_copy(ones_v, cacc.at[dst_v.at[pl.ds(0, E_CH)]],
                                  sem_sa).wait()

        gather(0, 0)

        def chunk2(p, carry):
            c = 2 * p
            gather(c + 1, 1)
            drain_g(0)
            scatter(c, 0)
            gather(c + 2, 0)
            drain_g(1)
            scatter(c + 1, 1)

            @pl.when(p > 0)
            def _():
                drain_ones()
                drain_ones()

            return carry

        lax.fori_loop(0, (N_CH - 3) // 2, chunk2, 0)
        gather(N_CH - 2, 1)
        drain_g(0)
        scatter(N_CH - 3, 0)
        gather(N_CH - 1, 0)
        drain_g(1)
        scatter(N_CH - 2, 1)
        drain_g(0)
        scatter(N_CH - 1, 0)
        for _ in range(5):
            drain_ones()

        plsc.subcore_barrier()

        for k in range(ROWS_TILE // E_CH):
            r0 = base + k * E_CH

            @pl.when(r0 + E_CH <= N_NODES)
            def _():
                pltpu.async_copy(acc.at[pl.ds(r0, E_CH)],
                                 psum_hbm.at[cid, pl.ds(r0, E_CH)], sem_gb)

        pltpu.sync_copy(cacc.at[pl.ds(base, ROWS_TILE)],
                        pcnt_hbm.at[cid, pl.ds(base, ROWS_TILE)])
        for k in range(ROWS_TILE // E_CH):
            r0 = base + k * E_CH

            @pl.when(r0 + E_CH <= N_NODES)
            def _():
                pltpu.make_async_copy(
                    acc.at[pl.ds(r0, E_CH)],
                    psum_hbm.at[cid, pl.ds(r0, E_CH)], sem_gb).wait()

    return agg(features, edges1d)


BLK = 2000


def _tc_finish(psum, cnt0, cnt1, features, W_relT, W_rootT, ln_w2, ln_b2):
    def body(ps_ref, pc0_ref, pc1_ref, x_ref, wr_ref, wt_ref, lnw_ref,
             lnb_ref, o_ref):
        s = ps_ref[0] + ps_ref[1]
        cnt = pc0_ref[...] + pc1_ref[...]
        mean = s / jnp.maximum(cnt, 1.0)
        out = jnp.dot(mean, wr_ref[...], preferred_element_type=jnp.float32)
        out = out + jnp.dot(x_ref[...], wt_ref[...],
                            preferred_element_type=jnp.float32)
        mu = jnp.mean(out, axis=-1, keepdims=True)
        d = out - mu
        var = jnp.mean(d * d, axis=-1, keepdims=True)
        normed = d * lax.rsqrt(var + 1e-5)
        o_ref[...] = jnp.maximum(normed * lnw_ref[...] + lnb_ref[...], 0.0)

    return pl.pallas_call(
        body,
        grid=(N_NODES // BLK,),
        in_specs=[
            pl.BlockSpec((NC, BLK, D), lambda i: (0, i, 0)),
            pl.BlockSpec((BLK, 1), lambda i: (i, 0)),
            pl.BlockSpec((BLK, 1), lambda i: (i, 0)),
            pl.BlockSpec((BLK, D), lambda i: (i, 0)),
            pl.BlockSpec((D, D), lambda i: (0, 0)),
            pl.BlockSpec((D, D), lambda i: (0, 0)),
            pl.BlockSpec((1, D), lambda i: (0, 0)),
            pl.BlockSpec((1, D), lambda i: (0, 0)),
        ],
        out_specs=pl.BlockSpec((BLK, D), lambda i: (i, 0)),
        out_shape=jax.ShapeDtypeStruct((N_NODES, D), jnp.float32),
    )(psum, cnt0, cnt1, features, W_relT, W_rootT, ln_w2, ln_b2)


def kernel(features, edges, W_rel, W_root, ln_w, ln_b):
    edges1d = jnp.concatenate(
        [edges.astype(jnp.int32).reshape(-1),
         jnp.zeros((E_FLAT - 2 * N_EDGES,), jnp.int32)])
    psum, pcnt = _sc_aggregate(features, edges1d)
    cnt0 = pcnt[0, :N_NODES].reshape(N_NODES, 1)
    cnt1 = pcnt[1, :N_NODES].reshape(N_NODES, 1)
    return _tc_finish(psum, cnt0, cnt1, features, W_rel.T, W_root.T,
                      ln_w.reshape(1, D), ln_b.reshape(1, D))


# speedup vs baseline: 1.0250x; 1.0250x over previous
import functools

import jax
import jax.numpy as jnp
from jax import lax
from jax.experimental import pallas as pl
from jax.experimental.pallas import tpu as pltpu
from jax.experimental.pallas import tpu_sc as plsc

N_NODES = 10000
N_EDGES = 320000
D = 128

NC = 2
NS = 16
NW = NC * NS
E_TILE = N_EDGES // NW
E_CH = 80
N_CH = E_TILE // E_CH
N_PAD = 10240
ROWS_TILE = N_PAD // NS
ST_LEN = 10112
E_FLAT = 2 * N_EDGES + 128


def _fill_f32(ref, rows, cols, value):
    per_row = cols // 16

    def body(t, carry):
        i = t // per_row
        j = t % per_row
        ref[i, pl.ds(j * 16, 16)] = jnp.full((16,), value, jnp.float32)
        return carry

    lax.fori_loop(0, rows * per_row, body, 0)


def _sc_aggregate(features, edges1d):
    mesh = plsc.VectorSubcoreMesh(core_axis_name="c", subcore_axis_name="s")

    @functools.partial(
        pl.kernel,
        mesh=mesh,
        out_type=[
            jax.ShapeDtypeStruct((NC, N_NODES, D), jnp.float32),
            jax.ShapeDtypeStruct((NC, N_PAD), jnp.float32),
        ],
        scratch_types=[
            pltpu.VMEM((ST_LEN,), jnp.int32),
            pltpu.VMEM((ST_LEN,), jnp.int32),
            pltpu.VMEM((2, E_CH, D), jnp.float32),
            pltpu.VMEM((E_CH,), jnp.float32),
            pltpu.VMEM_SHARED((N_PAD, D), jnp.float32),
            pltpu.VMEM_SHARED((N_PAD,), jnp.float32),
            pltpu.SemaphoreType.DMA,
            pltpu.SemaphoreType.DMA,
            pltpu.SemaphoreType.DMA,
            pltpu.SemaphoreType.DMA,
        ],
    )
    def agg(features_hbm, edges_hbm, psum_hbm, pcnt_hbm,
            src_v, dst_v, rows2, ones_v, acc, cacc,
            sem_ga, sem_gb, sem_sa, sem_sb):
        cid = lax.axis_index("c")
        sid = lax.axis_index("s")
        gid = cid * NS + sid
        base = sid * ROWS_TILE

        zbuf = rows2.at[0]
        _fill_f32(zbuf, E_CH, D, 0.0)
        for k in range(ROWS_TILE // E_CH):
            pltpu.sync_copy(zbuf, acc.at[pl.ds(base + k * E_CH, E_CH)])
        for k in range(ROWS_TILE // D):
            pltpu.sync_copy(zbuf.at[0], cacc.at[pl.ds(base + k * D, D)])

        def fill_ones(j, carry):
            ones_v[pl.ds(j * 16, 16)] = jnp.full((16,), 1.0, jnp.float32)
            return carry

        lax.fori_loop(0, E_CH // 16, fill_ones, 0)

        pltpu.sync_copy(edges_hbm.at[pl.ds(gid * E_TILE, ST_LEN)], src_v)
        pltpu.sync_copy(edges_hbm.at[pl.ds(N_EDGES + gid * E_TILE, ST_LEN)],
                        dst_v)

        plsc.subcore_barrier()

        gsem = (sem_ga, sem_gb)

        def gather(c, buf):
            pltpu.async_copy(
                features_hbm.at[src_v.at[pl.ds(c * E_CH, E_CH)]],
                rows2.at[buf], gsem[buf])

        def drain_g(buf):
            pltpu.make_async_copy(
                features_hbm.at[src_v.at[pl.ds(0, E_CH)]],
                rows2.at[buf], gsem[buf]).wait()

        def scatter(c, buf):
            idx = dst_v.at[pl.ds(c * E_CH, E_CH)]
            pltpu.sync_copy(rows2.at[buf], acc.at[idx], add=True)
            pltpu.async_copy(ones_v, cacc.at[idx], sem_sa, add=True)

        def drain_ones():
            pltpu.make_async_copy(ones_v, cacc.at[dst_v.at[pl.ds(0, E_CH)]],
                                  sem_sa).wait()

        gather(0, 0)

        def chunk2(p, carry):
            c = 2 * p
            gather(c + 1, 1)
            drain_g(0)
            scatter(c, 0)
            gather(c + 2, 0)
            drain_g(1)
            scatter(c + 1, 1)

            @pl.when(p > 0)
            def _():
                drain_ones()
                drain_ones()

            return carry

        lax.fori_loop(0, (N_CH - 3) // 2, chunk2, 0)
        gather(N_CH - 2, 1)
        drain_g(0)
        scatter(N_CH - 3, 0)
        gather(N_CH - 1, 0)
        drain_g(1)
        scatter(N_CH - 2, 1)
        drain_g(0)
        scatter(N_CH - 1, 0)
        for _ in range(5):
            drain_ones()

        plsc.subcore_barrier()

        for k in range(ROWS_TILE // E_CH):
            r0 = base + k * E_CH

            @pl.when(r0 + E_CH <= N_NODES)
            def _():
                pltpu.async_copy(acc.at[pl.ds(r0, E_CH)],
                                 psum_hbm.at[cid, pl.ds(r0, E_CH)], sem_gb)

        pltpu.sync_copy(cacc.at[pl.ds(base, ROWS_TILE)],
                        pcnt_hbm.at[cid, pl.ds(base, ROWS_TILE)])
        for k in range(ROWS_TILE // E_CH):
            r0 = base + k * E_CH

            @pl.when(r0 + E_CH <= N_NODES)
            def _():
                pltpu.make_async_copy(
                    acc.at[pl.ds(r0, E_CH)],
                    psum_hbm.at[cid, pl.ds(r0, E_CH)], sem_gb).wait()

    return agg(features, edges1d)


BLK = 2000


def _tc_finish(psum, cnt2, features, W_relT, W_rootT, ln_w2, ln_b2):
    def body(ps_ref, pc_ref, x_ref, wr_ref, wt_ref, lnw_ref, lnb_ref, o_ref):
        s = ps_ref[0] + ps_ref[1]
        cnt = jnp.sum(pc_ref[...], axis=1, keepdims=True)
        mean = s / jnp.maximum(cnt, 1.0)
        out = jnp.dot(mean, wr_ref[...], preferred_element_type=jnp.float32)
        out = out + jnp.dot(x_ref[...], wt_ref[...],
                            preferred_element_type=jnp.float32)
        mu = jnp.mean(out, axis=-1, keepdims=True)
        d = out - mu
        var = jnp.mean(d * d, axis=-1, keepdims=True)
        normed = d * lax.rsqrt(var + 1e-5)
        o_ref[...] = jnp.maximum(normed * lnw_ref[...] + lnb_ref[...], 0.0)

    return pl.pallas_call(
        body,
        grid=(N_NODES // BLK,),
        in_specs=[
            pl.BlockSpec((NC, BLK, D), lambda i: (0, i, 0)),
            pl.BlockSpec((BLK, NC), lambda i: (i, 0)),
            pl.BlockSpec((BLK, D), lambda i: (i, 0)),
            pl.BlockSpec((D, D), lambda i: (0, 0)),
            pl.BlockSpec((D, D), lambda i: (0, 0)),
            pl.BlockSpec((1, D), lambda i: (0, 0)),
            pl.BlockSpec((1, D), lambda i: (0, 0)),
        ],
        out_specs=pl.BlockSpec((BLK, D), lambda i: (i, 0)),
        out_shape=jax.ShapeDtypeStruct((N_NODES, D), jnp.float32),
    )(psum, cnt2, features, W_relT, W_rootT, ln_w2, ln_b2)


def kernel(features, edges, W_rel, W_root, ln_w, ln_b):
    edges1d = jnp.concatenate(
        [edges.astype(jnp.int32).reshape(-1),
         jnp.zeros((E_FLAT - 2 * N_EDGES,), jnp.int32)])
    psum, pcnt = _sc_aggregate(features, edges1d)
    cnt2 = jnp.stack([pcnt[0, :N_NODES], pcnt[1, :N_NODES]], axis=-1)
    return _tc_finish(psum, cnt2, features, W_rel.T, W_root.T,
                      ln_w.reshape(1, D), ln_b.reshape(1, D))


# speedup vs baseline: 1.0284x; 1.0033x over previous
import functools

import jax
import jax.numpy as jnp
from jax import lax
from jax.experimental import pallas as pl
from jax.experimental.pallas import tpu as pltpu
from jax.experimental.pallas import tpu_sc as plsc

N_NODES = 10000
N_EDGES = 320000
D = 128

NC = 2
NS = 16
NW = NC * NS
E_TILE = N_EDGES // NW
E_CH = 80
N_CH = E_TILE // E_CH
N_PAD = 10240
ROWS_TILE = N_PAD // NS
ST_LEN = 10112
E_FLAT = 2 * N_EDGES + 128


def _fill_f32(ref, rows, cols, value):
    per_row = cols // 16

    def body(t, carry):
        i = t // per_row
        j = t % per_row
        ref[i, pl.ds(j * 16, 16)] = jnp.full((16,), value, jnp.float32)
        return carry

    lax.fori_loop(0, rows * per_row, body, 0)


def _sc_aggregate(features, edges1d):
    mesh = plsc.VectorSubcoreMesh(core_axis_name="c", subcore_axis_name="s")

    @functools.partial(
        pl.kernel,
        mesh=mesh,
        out_type=[
            jax.ShapeDtypeStruct((NC, N_NODES, D), jnp.float32),
            jax.ShapeDtypeStruct((NC, N_PAD), jnp.float32),
        ],
        scratch_types=[
            pltpu.VMEM((ST_LEN,), jnp.int32),
            pltpu.VMEM((ST_LEN,), jnp.int32),
            pltpu.VMEM((2, E_CH, D), jnp.float32),
            pltpu.VMEM((128,), jnp.float32),
            pltpu.VMEM_SHARED((N_PAD, D), jnp.float32),
            pltpu.VMEM_SHARED((N_PAD,), jnp.float32),
            pltpu.SemaphoreType.DMA,
            pltpu.SemaphoreType.DMA,
            pltpu.SemaphoreType.DMA,
            pltpu.SemaphoreType.DMA,
        ],
    )
    def agg(features_hbm, edges_hbm, psum_hbm, pcnt_hbm,
            src_v, dst_v, rows2, ones_v, acc, cacc,
            sem_ga, sem_gb, sem_sa, sem_sb):
        cid = lax.axis_index("c")
        sid = lax.axis_index("s")
        gid = cid * NS + sid
        base = sid * ROWS_TILE

        zbuf = rows2.at[0]
        _fill_f32(zbuf, E_CH, D, 0.0)
        for k in range(ROWS_TILE // E_CH):
            pltpu.sync_copy(zbuf, acc.at[pl.ds(base + k * E_CH, E_CH)])
        for k in range(ROWS_TILE // D):
            pltpu.sync_copy(zbuf.at[0], cacc.at[pl.ds(base + k * D, D)])

        def fill_ones(j, carry):
            ones_v[pl.ds(j * 16, 16)] = jnp.full((16,), 1.0, jnp.float32)
            return carry

        lax.fori_loop(0, 128 // 16, fill_ones, 0)

        pltpu.sync_copy(edges_hbm.at[pl.ds(gid * E_TILE, ST_LEN)], src_v)
        pltpu.sync_copy(edges_hbm.at[pl.ds(N_EDGES + gid * E_TILE, ST_LEN)],
                        dst_v)

        plsc.subcore_barrier()

        gsem = (sem_ga, sem_gb)

        def gather(c, buf):
            pltpu.async_copy(
                features_hbm.at[src_v.at[pl.ds(c * E_CH, E_CH)]],
                rows2.at[buf], gsem[buf])

        def drain_g(buf):
            pltpu.make_async_copy(
                features_hbm.at[src_v.at[pl.ds(0, E_CH)]],
                rows2.at[buf], gsem[buf]).wait()

        def scatter(c, buf):
            idx = dst_v.at[pl.ds(c * E_CH, E_CH)]
            pltpu.sync_copy(rows2.at[buf], acc.at[idx], add=True)

        def ones_issue(o):
            idx = dst_v.at[pl.ds(o * 128, 128)]
            pltpu.async_copy(ones_v, cacc.at[idx], sem_sa, add=True)

        def drain_ones():
            pltpu.make_async_copy(ones_v, cacc.at[dst_v.at[pl.ds(0, 128)]],
                                  sem_sa).wait()

        gather(0, 0)

        def chunk2(p, carry):
            c = 2 * p
            gather(c + 1, 1)
            drain_g(0)
            scatter(c, 0)
            gather(c + 2, 0)
            drain_g(1)
            scatter(c + 1, 1)

            @pl.when(p < 39)
            def _():
                ones_issue(2 * p)
                ones_issue(2 * p + 1)

            @pl.when((p > 0) & (p < 40))
            def _():
                drain_ones()
                drain_ones()

            return carry

        lax.fori_loop(0, (N_CH - 3) // 2, chunk2, 0)
        gather(N_CH - 2, 1)
        drain_g(0)
        scatter(N_CH - 3, 0)
        gather(N_CH - 1, 0)
        drain_g(1)
        scatter(N_CH - 2, 1)
        drain_g(0)
        scatter(N_CH - 1, 0)
        idx16 = dst_v.at[pl.ds(78 * 128, 16)]
        pltpu.async_copy(ones_v.at[pl.ds(0, 16)], cacc.at[idx16], sem_sa,
                         add=True)
        pltpu.make_async_copy(ones_v.at[pl.ds(0, 16)], cacc.at[idx16],
                              sem_sa).wait()

        plsc.subcore_barrier()

        for k in range(ROWS_TILE // E_CH):
            r0 = base + k * E_CH

            @pl.when(r0 + E_CH <= N_NODES)
            def _():
                pltpu.async_copy(acc.at[pl.ds(r0, E_CH)],
                                 psum_hbm.at[cid, pl.ds(r0, E_CH)], sem_gb)

        pltpu.sync_copy(cacc.at[pl.ds(base, ROWS_TILE)],
                        pcnt_hbm.at[cid, pl.ds(base, ROWS_TILE)])
        for k in range(ROWS_TILE // E_CH):
            r0 = base + k * E_CH

            @pl.when(r0 + E_CH <= N_NODES)
            def _():
                pltpu.make_async_copy(
                    acc.at[pl.ds(r0, E_CH)],
                    psum_hbm.at[cid, pl.ds(r0, E_CH)], sem_gb).wait()

    return agg(features, edges1d)


BLK = 2000


def _tc_finish(psum, cnt2, features, W_relT, W_rootT, ln_w2, ln_b2):
    def body(ps_ref, pc_ref, x_ref, wr_ref, wt_ref, lnw_ref, lnb_ref, o_ref):
        s = ps_ref[0] + ps_ref[1]
        cnt = jnp.sum(pc_ref[...], axis=1, keepdims=True)
        mean = s / jnp.maximum(cnt, 1.0)
        out = jnp.dot(mean, wr_ref[...], preferred_element_type=jnp.float32)
        out = out + jnp.dot(x_ref[...], wt_ref[...],
                            preferred_element_type=jnp.float32)
        mu = jnp.mean(out, axis=-1, keepdims=True)
        d = out - mu
        var = jnp.mean(d * d, axis=-1, keepdims=True)
        normed = d * lax.rsqrt(var + 1e-5)
        o_ref[...] = jnp.maximum(normed * lnw_ref[...] + lnb_ref[...], 0.0)

    return pl.pallas_call(
        body,
        grid=(N_NODES // BLK,),
        in_specs=[
            pl.BlockSpec((NC, BLK, D), lambda i: (0, i, 0)),
            pl.BlockSpec((BLK, NC), lambda i: (i, 0)),
            pl.BlockSpec((BLK, D), lambda i: (i, 0)),
            pl.BlockSpec((D, D), lambda i: (0, 0)),
            pl.BlockSpec((D, D), lambda i: (0, 0)),
            pl.BlockSpec((1, D), lambda i: (0, 0)),
            pl.BlockSpec((1, D), lambda i: (0, 0)),
        ],
        out_specs=pl.BlockSpec((BLK, D), lambda i: (i, 0)),
        out_shape=jax.ShapeDtypeStruct((N_NODES, D), jnp.float32),
    )(psum, cnt2, features, W_relT, W_rootT, ln_w2, ln_b2)


def kernel(features, edges, W_rel, W_root, ln_w, ln_b):
    edges1d = jnp.concatenate(
        [edges.astype(jnp.int32).reshape(-1),
         jnp.zeros((E_FLAT - 2 * N_EDGES,), jnp.int32)])
    psum, pcnt = _sc_aggregate(features, edges1d)
    cnt2 = jnp.stack([pcnt[0, :N_NODES], pcnt[1, :N_NODES]], axis=-1)
    return _tc_finish(psum, cnt2, features, W_rel.T, W_root.T,
                      ln_w.reshape(1, D), ln_b.reshape(1, D))
